# baseline (speedup 1.0000x reference)
import functools

import jax
import jax.numpy as jnp
from jax import lax
from jax.experimental import pallas as pl
from jax.experimental.pallas import tpu as pltpu
from jax.experimental.pallas import tpu_sc as plsc

_VOCAB = 1000000
_EMBED = 64
_BATCH = 16384
_HIST = 50
_NW = 32
_BPW = _BATCH // _NW
_CB = 256
_GW = 128
_KG = _CB // _GW
_NCH = _HIST * (_BPW // _CB)
_L = 16


def _sc_gather(idxt_hbm, table_hbm, out_hbm, idx_v, rows_a, rows_b, tr_a,
               tr_b, gsem_a, gsem_b, osem_a, osem_b):
    wid = lax.axis_index("s") * 2 + lax.axis_index("c")
    b0 = pl.multiple_of(wid * _BPW, _BPW)

    pltpu.sync_copy(idxt_hbm.at[:, pl.ds(b0, _BPW)], idx_v)

    def coords(c):
        h = c // 2
        boff = pl.multiple_of((c % 2) * _CB, _CB)
        return h, boff

    def fire_gathers(c, rows_v, sem):
        h, boff = coords(c)
        return [
            pltpu.async_copy(
                table_hbm.at[idx_v.at[h, pl.ds(boff + j * _GW, _GW)]],
                rows_v.at[pl.ds(j * _GW, _GW)],
                sem,
            )
            for j in range(_KG)
        ]

    def drain_gathers(rows_v, sem):
        for j in range(_KG):
            pltpu.make_async_copy(
                table_hbm.at[idx_v.at[0, pl.ds(j * _GW, _GW)]],
                rows_v.at[pl.ds(j * _GW, _GW)],
                sem,
            ).wait()

    def transpose(rows_v, tr_v):
        @plsc.parallel_loop(0, _CB, unroll=4)
        def per_b(b):
            bvec = jnp.full((_L,), b, dtype=jnp.int32)
            for m in range(_EMBED // _L):
                evec = jnp.arange(_L, dtype=jnp.int32) + (m * _L)
                plsc.store_scatter(
                    tr_v, [evec, bvec], rows_v[b, pl.ds(m * _L, _L)])

    def fire_store(c, tr_v, sem):
        h, boff = coords(c)
        return pltpu.async_copy(
            tr_v, out_hbm.at[h, :, pl.ds(b0 + boff, _CB)], sem)

    def wait_store(tr_v, sem):
        pltpu.make_async_copy(
            tr_v, out_hbm.at[0, :, pl.ds(b0, _CB)], sem).wait()

    fire_gathers(0, rows_a, gsem_a)
    drain_gathers(rows_a, gsem_a)
    fire_gathers(1, rows_b, gsem_b)
    transpose(rows_a, tr_a)
    fire_store(0, tr_a, osem_a)

    def body(p, carry):
        c = 2 * p + 1
        drain_gathers(rows_b, gsem_b)
        fire_gathers(c + 1, rows_a, gsem_a)
        transpose(rows_b, tr_b)
        wait_store(tr_a, osem_a)
        fire_store(c, tr_b, osem_b)
        drain_gathers(rows_a, gsem_a)
        fire_gathers(c + 2, rows_b, gsem_b)
        transpose(rows_a, tr_a)
        wait_store(tr_b, osem_b)
        fire_store(c + 1, tr_a, osem_a)
        return carry

    lax.fori_loop(0, _NCH // 2 - 1, body, 0)

    drain_gathers(rows_b, gsem_b)
    transpose(rows_b, tr_b)
    wait_store(tr_a, osem_a)
    fire_store(_NCH - 1, tr_b, osem_b)
    wait_store(tr_b, osem_b)


_mesh = plsc.VectorSubcoreMesh(core_axis_name="c", subcore_axis_name="s")

_gather_call = functools.partial(
    pl.kernel,
    out_type=jax.ShapeDtypeStruct((_HIST, _EMBED, _BATCH), jnp.float32),
    mesh=_mesh,
    compiler_params=pltpu.CompilerParams(
        use_tc_tiling_on_sc=False, needs_layout_passes=False),
    scratch_types=[
        pltpu.VMEM((_HIST, _BPW), jnp.int32),
        pltpu.VMEM((_CB, _EMBED), jnp.float32),
        pltpu.VMEM((_CB, _EMBED), jnp.float32),
        pltpu.VMEM((_EMBED, _CB), jnp.float32),
        pltpu.VMEM((_EMBED, _CB), jnp.float32),
        pltpu.SemaphoreType.DMA,
        pltpu.SemaphoreType.DMA,
        pltpu.SemaphoreType.DMA,
        pltpu.SemaphoreType.DMA,
    ],
)(_sc_gather)


@jax.jit
def kernel(input, weight):
    out_t = _gather_call(input.T.astype(jnp.int32), weight)
    return jnp.transpose(out_t, (2, 0, 1))


# speedup vs baseline: 1.2076x; 1.2076x over previous
import functools

import jax
import jax.numpy as jnp
from jax import lax
from jax.experimental import pallas as pl
from jax.experimental.pallas import tpu as pltpu
from jax.experimental.pallas import tpu_sc as plsc

_VOCAB = 1000000
_EMBED = 64
_BATCH = 16384
_HIST = 50
_NTOT = _BATCH * _HIST
_NW = 32
_RPW = _NTOT // _NW
_GW = 128
_C = 512
_KG = _C // _GW
_NCH = _RPW // _C
_IDXROWS = _RPW // _GW


def _sc_gather(idx_hbm, table_hbm, out_hbm, idx_v, rows_a, rows_b, gsem_a,
               gsem_b, osem_a, osem_b):
    wid = lax.axis_index("s") * 2 + lax.axis_index("c")
    base = wid * _RPW

    pltpu.sync_copy(
        idx_hbm.at[pl.ds(pl.multiple_of(wid * _IDXROWS, 8), _IDXROWS)], idx_v)

    def fire_gathers(g, rows_v, sem):
        return [
            pltpu.async_copy(
                table_hbm.at[idx_v.at[g * _KG + j]],
                rows_v.at[pl.ds(j * _GW, _GW)],
                sem,
            )
            for j in range(_KG)
        ]

    def drain_gathers(rows_v, sem):
        for j in range(_KG):
            pltpu.make_async_copy(
                table_hbm.at[idx_v.at[j]],
                rows_v.at[pl.ds(j * _GW, _GW)],
                sem,
            ).wait()

    def store(g, rows_v, sem):
        return pltpu.async_copy(
            rows_v, out_hbm.at[pl.ds(pl.multiple_of(base + g * _C, _C), _C)],
            sem)

    def wait_store(g, rows_v, sem):
        pltpu.make_async_copy(
            rows_v, out_hbm.at[pl.ds(pl.multiple_of(base + g * _C, _C), _C)],
            sem).wait()

    fire_gathers(0, rows_a, gsem_a)
    drain_gathers(rows_a, gsem_a)
    fire_gathers(1, rows_b, gsem_b)
    store(0, rows_a, osem_a)

    def body(p, carry):
        g = 2 * p + 1
        drain_gathers(rows_b, gsem_b)
        wait_store(g - 1, rows_a, osem_a)
        fire_gathers(g + 1, rows_a, gsem_a)
        store(g, rows_b, osem_b)
        drain_gathers(rows_a, gsem_a)
        wait_store(g, rows_b, osem_b)
        fire_gathers(g + 2, rows_b, gsem_b)
        store(g + 1, rows_a, osem_a)
        return carry

    lax.fori_loop(0, _NCH // 2 - 1, body, 0)

    drain_gathers(rows_b, gsem_b)
    store(_NCH - 1, rows_b, osem_b)
    wait_store(_NCH - 2, rows_a, osem_a)
    wait_store(_NCH - 1, rows_b, osem_b)


_mesh = plsc.VectorSubcoreMesh(core_axis_name="c", subcore_axis_name="s")

_gather_call = functools.partial(
    pl.kernel,
    out_type=jax.ShapeDtypeStruct((_NTOT, _EMBED), jnp.float32),
    mesh=_mesh,
    compiler_params=pltpu.CompilerParams(use_tc_tiling_on_sc=False),
    scratch_types=[
        pltpu.VMEM((_IDXROWS, _GW), jnp.int32),
        pltpu.VMEM((_C, _EMBED), jnp.float32),
        pltpu.VMEM((_C, _EMBED), jnp.float32),
        pltpu.SemaphoreType.DMA,
        pltpu.SemaphoreType.DMA,
        pltpu.SemaphoreType.DMA,
        pltpu.SemaphoreType.DMA,
    ],
)(_sc_gather)


@jax.jit
def kernel(input, weight):
    idx = input.reshape(_NTOT // _GW, _GW).astype(jnp.int32)
    rows = _gather_call(idx, weight)
    return rows.reshape(_BATCH, _HIST, _EMBED)
